# baseline (speedup 1.0000x reference)
import functools

import jax
import jax.numpy as jnp
import numpy as np
from jax import lax
from jax.experimental import pallas as pl
from jax.experimental.pallas import tpu as pltpu
from jax.experimental.pallas import tpu_sc as plsc

_KW = float(np.sqrt(3.0 * 3.0 * np.pi))
_N_LORS = 50000

_NC, _NS = 2, 16
_NW = _NC * _NS
_N_PAD = 150528
_PER_W = _N_PAD // _NW
_GROUPS = _PER_W // 16
_HALF = _PER_W // 2

_S_MAJ, _S_MID = 16384, 128
_IMG_WORDS = 128 * 128 * 64
_CHUNK = _IMG_WORDS // _NS

_INV_VOX = np.float32(1.0 / 3.125)
_COFF = np.float32(63.5)
_INV63 = np.float32(1.0 / 63.0)
_OSCALE = np.float32(1.0 / (64.0 * _KW * 255.0))


def _body(img_hbm, a1h, b1h, c1h, a2h, b2h, c2h, out_hbm,
          img_s, a1v, b1v, c1v, a2v, b2v, c2v,
          idx0, idx1, got0, got1,
          fa0, fa1, fb0, fb1, fc0, fc1, pr0, pr1,
          out_v, sem0, sem1):
    c = lax.axis_index("c")
    s = lax.axis_index("s")
    wid = c * _NS + s
    base = pl.multiple_of(wid * _PER_W, 8)

    cols_hbm = (a1h, b1h, c1h, a2h, b2h, c2h)
    lor_bufs = (a1v, b1v, c1v, a2v, b2v, c2v)
    idx_b = (idx0, idx1)
    got_b = (got0, got1)
    fa_b, fb_b, fc_b, pr_b = (fa0, fa1), (fb0, fb1), (fc0, fc1), (pr0, pr1)
    sem_b = (sem0, sem1)

    off = pl.multiple_of(s * _CHUNK, 8)
    pltpu.sync_copy(img_hbm.at[pl.ds(off, _CHUNK)], img_s.at[pl.ds(off, _CHUNK)])

    def stage_lors(hoff):
        for hb, vm in zip(cols_hbm, lor_bufs):
            pltpu.sync_copy(hb.at[pl.ds(base + hoff, _HALF)], vm)

    stage_lors(0)
    plsc.subcore_barrier()

    def preamble_phase1(g, b):
        gb = g * 16 - jnp.where(g >= _GROUPS // 2, _HALF, 0)
        idx_v, fa_v, fb_v, fc_v, par_v = idx_b[b], fa_b[b], fb_b[b], fc_b[b], pr_b[b]
        p1a = a1v[pl.ds(gb, 16)]
        p1b = b1v[pl.ds(gb, 16)]
        p1c = c1v[pl.ds(gb, 16)]
        da = a2v[pl.ds(gb, 16)] - p1a
        db = b2v[pl.ds(gb, 16)] - p1b
        dc = c2v[pl.ds(gb, 16)] - p1c
        s2 = jnp.maximum(da * da + db * db + dc * dc, np.float32(1e-30))
        seed = lax.bitcast_convert_type(
            (lax.bitcast_convert_type(s2, jnp.int32) >> 1) + 0x1FBD1DF5,
            jnp.float32)
        half = np.float32(0.5)
        y = half * (seed + s2 / seed)
        length = half * (y + s2 / y)
        scale = length * _OSCALE

        va0 = p1a * _INV_VOX + _COFF
        vb0 = p1b * _INV_VOX + _COFF
        vc0 = p1c * _INV_VOX + _COFF
        dva = da * (_INV_VOX * _INV63)
        dvb = db * (_INV_VOX * _INV63)
        dvc = dc * (_INV_VOX * _INV63)

        def step1(i, carry1):
            ua, ub, uc = carry1
            ia = ua.astype(jnp.int32)
            ib = ub.astype(jnp.int32)
            ic = uc.astype(jnp.int32)
            fa_v[pl.ds(i * 16, 16)] = ua - ia.astype(jnp.float32)
            fb_v[pl.ds(i * 16, 16)] = ub - ib.astype(jnp.float32)
            fc_v[pl.ds(i * 16, 16)] = uc - ic.astype(jnp.float32)
            lin = (ia << 14) + (ib << 7) + ic
            par_v[pl.ds(i * 16, 16)] = lin & 1
            ibase = i * 64
            for k, o in enumerate((0, _S_MID, _S_MAJ, _S_MAJ + _S_MID)):
                idx_v[pl.ds(ibase + k * 16, 16)] = (lin + o) >> 1
            return (ua + dva, ub + dvb, uc + dvc)

        lax.fori_loop(0, 64, step1, (va0, vb0, vc0), unroll=4)
        return scale

    def fire(b):
        for r in range(4):
            pltpu.async_copy(
                img_s.at[idx_b[b].at[pl.ds(r * 1024, 1024)]],
                got_b[b].at[pl.ds(r * 1024, 1024)],
                sem_b[b],
            )

    def drain(b):
        pltpu.make_async_copy(
            img_hbm.at[pl.ds(0, 64 * 64)], got_b[b], sem_b[b]).wait()

    def phase3(g, b, scale):
        got_v, fa_v, fb_v, fc_v, par_v = got_b[b], fa_b[b], fb_b[b], fc_b[b], pr_b[b]

        def step2(i, acc):
            ibase = i * 64
            fb16 = i * 16
            fa = fa_v[pl.ds(fb16, 16)]
            fb = fb_v[pl.ds(fb16, 16)]
            fc = fc_v[pl.ds(fb16, 16)]
            sh = par_v[pl.ds(fb16, 16)] << 4

            def pairval(k):
                w = got_v[pl.ds(ibase + k * 16, 16)]
                u = w >> sh
                zlo = (u & 255).astype(jnp.float32)
                zhi = ((u >> 8) & 255).astype(jnp.float32)
                return zlo + fc * (zhi - zlo)

            v00 = pairval(0)
            v01 = pairval(1)
            v10 = pairval(2)
            v11 = pairval(3)
            r0 = v00 + fb * (v01 - v00)
            r1 = v10 + fb * (v11 - v10)
            return acc + (r0 + fa * (r1 - r0))

        acc = lax.fori_loop(0, 64, step2, jnp.zeros((16,), jnp.float32),
                            unroll=4)
        out_v[pl.ds(g * 16, 16)] = acc * scale

    scale0 = preamble_phase1(0, 0)
    fire(0)

    def pair(k, scale_c):
        for b in (0, 1):
            g = 2 * k + b
            gn = jnp.where(g + 1 >= _GROUPS, 0, g + 1)

            @pl.when(gn == _GROUPS // 2)
            def _():
                stage_lors(_HALF)

            scale_n = preamble_phase1(gn, b ^ 1)
            fire(b ^ 1)
            drain(b)
            phase3(g, b, scale_c)
            scale_c = scale_n
        return scale_c

    lax.fori_loop(0, _GROUPS // 2, pair, scale0, unroll=False)
    drain(0)

    pltpu.sync_copy(out_v, out_hbm.at[pl.ds(base, _PER_W)])


@functools.partial(jax.jit, static_argnums=())
def kernel(image, xlors, ylors, zlors):
    qf = jnp.floor(image.reshape(16384, 128) * np.float32(255.0)
                   + np.float32(0.5))
    s_even = np.zeros((128, 64), np.float32)
    s_even[2 * np.arange(64), np.arange(64)] = 1.0
    s_odd = np.zeros((128, 64), np.float32)
    s_odd[2 * np.arange(64) + 1, np.arange(64)] = 1.0
    a = (qf @ s_even).reshape(-1).astype(jnp.int32)
    bq = (qf @ s_odd).reshape(-1).astype(jnp.int32)
    cq = jnp.concatenate([a[1:], a[-1:]])
    img_w = a | (bq << 8) | (bq << 16) | (cq << 24)

    perm = [2, 0, 1, 5, 3, 4]
    oh = np.eye(6, dtype=np.float32)
    zpad = jnp.zeros((_N_PAD - 3 * _N_LORS,), jnp.float32)
    def mv(m, v):
        return jnp.dot(m, v, precision=lax.Precision.HIGHEST)

    cols = [
        jnp.concatenate([mv(xlors, oh[:, perm[j]]), mv(ylors, oh[:, perm[j]]),
                         mv(zlors, oh[:, j]), zpad])
        for j in range(6)
    ]

    run = pl.kernel(
        _body,
        out_type=jax.ShapeDtypeStruct((_N_PAD,), jnp.float32),
        mesh=plsc.VectorSubcoreMesh(core_axis_name="c", subcore_axis_name="s",
                                    num_cores=_NC, num_subcores=_NS),
        scratch_types=[
            pltpu.VMEM_SHARED((_IMG_WORDS,), jnp.int32),
            pltpu.VMEM((_HALF,), jnp.float32),
            pltpu.VMEM((_HALF,), jnp.float32),
            pltpu.VMEM((_HALF,), jnp.float32),
            pltpu.VMEM((_HALF,), jnp.float32),
            pltpu.VMEM((_HALF,), jnp.float32),
            pltpu.VMEM((_HALF,), jnp.float32),
            pltpu.VMEM((64 * 64,), jnp.int32),
            pltpu.VMEM((64 * 64,), jnp.int32),
            pltpu.VMEM((64 * 64,), jnp.int32),
            pltpu.VMEM((64 * 64,), jnp.int32),
            pltpu.VMEM((64 * 16,), jnp.float32),
            pltpu.VMEM((64 * 16,), jnp.float32),
            pltpu.VMEM((64 * 16,), jnp.float32),
            pltpu.VMEM((64 * 16,), jnp.float32),
            pltpu.VMEM((64 * 16,), jnp.float32),
            pltpu.VMEM((64 * 16,), jnp.float32),
            pltpu.VMEM((64 * 16,), jnp.int32),
            pltpu.VMEM((64 * 16,), jnp.int32),
            pltpu.VMEM((_PER_W,), jnp.float32),
            pltpu.SemaphoreType.DMA,
            pltpu.SemaphoreType.DMA,
        ],
    )
    out = run(img_w, *cols)
    return out[:_N_LORS], out[_N_LORS:2 * _N_LORS], out[2 * _N_LORS:3 * _N_LORS]


# speedup vs baseline: 1.0512x; 1.0512x over previous
import functools

import jax
import jax.numpy as jnp
import numpy as np
from jax import lax
from jax.experimental import pallas as pl
from jax.experimental.pallas import tpu as pltpu
from jax.experimental.pallas import tpu_sc as plsc

_KW = float(np.sqrt(3.0 * 3.0 * np.pi))
_N_LORS = 50000

_NC, _NS = 2, 16
_NW = _NC * _NS
_N_PAD = 150528
_PER_W = _N_PAD // _NW
_GROUPS = _PER_W // 16
_HALF = _PER_W // 2

_S_MAJ, _S_MID = 16384, 128
_IMG_WORDS = 128 * 128 * 64
_CHUNK = _IMG_WORDS // _NS

_INV_VOX = np.float32(1.0 / 3.125)
_COFF = np.float32(63.5)
_INV63 = np.float32(1.0 / 63.0)
_OSCALE = np.float32(1.0 / (64.0 * _KW * 255.0))


def _body(img_hbm, a1h, b1h, c1h, a2h, b2h, c2h, out_hbm,
          img_s, a1v, b1v, c1v, a2v, b2v, c2v,
          idx0, idx1, got0, got1,
          fa0, fa1, fb0, fb1, fc0, fc1, pr0, pr1,
          out_v, sem0, sem1):
    c = lax.axis_index("c")
    s = lax.axis_index("s")
    wid = c * _NS + s
    base = pl.multiple_of(wid * _PER_W, 8)

    cols_hbm = (a1h, b1h, c1h, a2h, b2h, c2h)
    lor_bufs = (a1v, b1v, c1v, a2v, b2v, c2v)
    idx_b = (idx0, idx1)
    got_b = (got0, got1)
    fa_b, fb_b, fc_b, pr_b = (fa0, fa1), (fb0, fb1), (fc0, fc1), (pr0, pr1)
    sem_b = (sem0, sem1)

    off = pl.multiple_of(s * _CHUNK, 8)
    pltpu.sync_copy(img_hbm.at[pl.ds(off, _CHUNK)], img_s.at[pl.ds(off, _CHUNK)])

    def stage_lors(hoff):
        for hb, vm in zip(cols_hbm, lor_bufs):
            pltpu.sync_copy(hb.at[pl.ds(base + hoff, _HALF)], vm)

    stage_lors(0)
    plsc.subcore_barrier()

    def preamble_phase1(g, b):
        gb = g * 16 - jnp.where(g >= _GROUPS // 2, _HALF, 0)
        idx_v, fa_v, fb_v, fc_v, par_v = idx_b[b], fa_b[b], fb_b[b], fc_b[b], pr_b[b]
        p1a = a1v[pl.ds(gb, 16)]
        p1b = b1v[pl.ds(gb, 16)]
        p1c = c1v[pl.ds(gb, 16)]
        da = a2v[pl.ds(gb, 16)] - p1a
        db = b2v[pl.ds(gb, 16)] - p1b
        dc = c2v[pl.ds(gb, 16)] - p1c
        s2 = jnp.maximum(da * da + db * db + dc * dc, np.float32(1e-30))
        seed = lax.bitcast_convert_type(
            (lax.bitcast_convert_type(s2, jnp.int32) >> 1) + 0x1FBD1DF5,
            jnp.float32)
        half = np.float32(0.5)
        y = half * (seed + s2 / seed)
        length = half * (y + s2 / y)
        scale = length * _OSCALE

        va0 = p1a * _INV_VOX + _COFF
        vb0 = p1b * _INV_VOX + _COFF
        vc0 = p1c * _INV_VOX + _COFF
        dva = da * (_INV_VOX * _INV63)
        dvb = db * (_INV_VOX * _INV63)
        dvc = dc * (_INV_VOX * _INV63)

        def step1(i, carry1):
            ua, ub, uc = carry1
            ia = ua.astype(jnp.int32)
            ib = ub.astype(jnp.int32)
            ic = uc.astype(jnp.int32)
            fa_v[pl.ds(i * 16, 16)] = ua - ia.astype(jnp.float32)
            fb_v[pl.ds(i * 16, 16)] = ub - ib.astype(jnp.float32)
            fc_v[pl.ds(i * 16, 16)] = uc - ic.astype(jnp.float32)
            lin = (ia << 14) + (ib << 7) + ic
            par_v[pl.ds(i * 16, 16)] = lin & 1
            ibase = i * 64
            for k, o in enumerate((0, _S_MID, _S_MAJ, _S_MAJ + _S_MID)):
                idx_v[pl.ds(ibase + k * 16, 16)] = (lin + o) >> 1
            return (ua + dva, ub + dvb, uc + dvc)

        lax.fori_loop(0, 64, step1, (va0, vb0, vc0), unroll=4)
        return scale

    def fire(b):
        for r in range(4):
            pltpu.async_copy(
                img_s.at[idx_b[b].at[pl.ds(r * 1024, 1024)]],
                got_b[b].at[pl.ds(r * 1024, 1024)],
                sem_b[b],
            )

    def drain(b):
        pltpu.make_async_copy(
            img_hbm.at[pl.ds(0, 64 * 64)], got_b[b], sem_b[b]).wait()

    def phase3(g, b, scale):
        got_v, fa_v, fb_v, fc_v, par_v = got_b[b], fa_b[b], fb_b[b], fc_b[b], pr_b[b]

        def step2(i, acc):
            ibase = i * 64
            fb16 = i * 16
            fa = fa_v[pl.ds(fb16, 16)]
            fb = fb_v[pl.ds(fb16, 16)]
            fc = fc_v[pl.ds(fb16, 16)]
            sh = par_v[pl.ds(fb16, 16)] << 4

            def pairval(k):
                w = got_v[pl.ds(ibase + k * 16, 16)]
                u = w >> sh
                zlo = (u & 255).astype(jnp.float32)
                zhi = ((u >> 8) & 255).astype(jnp.float32)
                return zlo + fc * (zhi - zlo)

            v00 = pairval(0)
            v01 = pairval(1)
            v10 = pairval(2)
            v11 = pairval(3)
            r0 = v00 + fb * (v01 - v00)
            r1 = v10 + fb * (v11 - v10)
            return acc + (r0 + fa * (r1 - r0))

        acc = lax.fori_loop(0, 64, step2, jnp.zeros((16,), jnp.float32),
                            unroll=4)
        out_v[pl.ds(g * 16, 16)] = acc * scale

    scale0 = preamble_phase1(0, 0)
    fire(0)

    def pair(k, scale_c):
        for b in (0, 1):
            g = 2 * k + b
            gn = jnp.where(g + 1 >= _GROUPS, 0, g + 1)

            @pl.when(gn == _GROUPS // 2)
            def _():
                stage_lors(_HALF)

            scale_n = preamble_phase1(gn, b ^ 1)
            fire(b ^ 1)
            drain(b)
            phase3(g, b, scale_c)
            scale_c = scale_n
        return scale_c

    lax.fori_loop(0, _GROUPS // 2, pair, scale0, unroll=False)
    drain(0)

    pltpu.sync_copy(out_v, out_hbm.at[pl.ds(base, _PER_W)])


@functools.partial(jax.jit, static_argnums=())
def kernel(image, xlors, ylors, zlors):
    qf = jnp.floor(image.reshape(16384, 128) * np.float32(255.0)
                   + np.float32(0.5))
    s_even = np.zeros((128, 64), np.float32)
    s_even[2 * np.arange(64), np.arange(64)] = 1.0
    s_odd = np.zeros((128, 64), np.float32)
    s_odd[2 * np.arange(64) + 1, np.arange(64)] = 1.0
    a = (qf @ s_even).reshape(-1).astype(jnp.int32)
    bq = (qf @ s_odd).reshape(-1).astype(jnp.int32)
    cq = jnp.concatenate([a[1:], a[-1:]])
    img_w = a | (bq << 8) | (bq << 16) | (cq << 24)

    perm = jnp.array([2, 0, 1, 5, 3, 4], dtype=jnp.int32)
    lall = jnp.concatenate([xlors[:, perm], ylors[:, perm], zlors], axis=0)
    lall = jnp.pad(lall, ((0, _N_PAD - 3 * _N_LORS), (0, 0)))
    cols = [lall[:, j] for j in range(6)]

    run = pl.kernel(
        _body,
        out_type=jax.ShapeDtypeStruct((_N_PAD,), jnp.float32),
        mesh=plsc.VectorSubcoreMesh(core_axis_name="c", subcore_axis_name="s",
                                    num_cores=_NC, num_subcores=_NS),
        scratch_types=[
            pltpu.VMEM_SHARED((_IMG_WORDS,), jnp.int32),
            pltpu.VMEM((_HALF,), jnp.float32),
            pltpu.VMEM((_HALF,), jnp.float32),
            pltpu.VMEM((_HALF,), jnp.float32),
            pltpu.VMEM((_HALF,), jnp.float32),
            pltpu.VMEM((_HALF,), jnp.float32),
            pltpu.VMEM((_HALF,), jnp.float32),
            pltpu.VMEM((64 * 64,), jnp.int32),
            pltpu.VMEM((64 * 64,), jnp.int32),
            pltpu.VMEM((64 * 64,), jnp.int32),
            pltpu.VMEM((64 * 64,), jnp.int32),
            pltpu.VMEM((64 * 16,), jnp.float32),
            pltpu.VMEM((64 * 16,), jnp.float32),
            pltpu.VMEM((64 * 16,), jnp.float32),
            pltpu.VMEM((64 * 16,), jnp.float32),
            pltpu.VMEM((64 * 16,), jnp.float32),
            pltpu.VMEM((64 * 16,), jnp.float32),
            pltpu.VMEM((64 * 16,), jnp.int32),
            pltpu.VMEM((64 * 16,), jnp.int32),
            pltpu.VMEM((_PER_W,), jnp.float32),
            pltpu.SemaphoreType.DMA,
            pltpu.SemaphoreType.DMA,
        ],
    )
    out = run(img_w, *cols)
    return out[:_N_LORS], out[_N_LORS:2 * _N_LORS], out[2 * _N_LORS:3 * _N_LORS]
